# baseline (speedup 1.0000x reference)
import functools

import jax
import jax.numpy as jnp
from jax import lax
from jax.experimental import pallas as pl
from jax.experimental.pallas import tpu as pltpu
from jax.experimental.pallas import tpu_sc as plsc

B = 16384
D = 32
NC = 2
NS = 16
L = 16
NW = NC * NS
BPW = B // NW
CHB = 128
CH = BPW // CHB


def _mf_body(user_hbm, item_hbm, ut_hbm, it_hbm, out_hbm,
             idx_u, idx_i, rows_u, rows_i, out_v, sem):
    wid = lax.axis_index("s") * NC + lax.axis_index("c")
    base = wid * BPW

    for c in range(CH):
        pltpu.sync_copy(user_hbm.at[pl.ds(base + c * CHB, CHB)], idx_u.at[c])
        pltpu.sync_copy(item_hbm.at[pl.ds(base + c * CHB, CHB)], idx_i.at[c])

    copies = []
    for c in range(CH):
        copies.append(pltpu.async_copy(
            ut_hbm.at[idx_u.at[c]], rows_u.at[pl.ds(c * CHB, CHB)], sem))
        copies.append(pltpu.async_copy(
            it_hbm.at[idx_i.at[c]], rows_i.at[pl.ds(c * CHB, CHB)], sem))
    for cp in copies:
        cp.wait()

    lane = lax.iota(jnp.int32, L)

    def group(g, carry):
        rows = g * L + lane
        acc = jnp.zeros((L,), jnp.float32)
        for d in range(D):
            col = jnp.full((L,), d, jnp.int32)
            u_d = plsc.load_gather(rows_u, [rows, col])
            v_d = plsc.load_gather(rows_i, [rows, col])
            acc = acc + u_d * v_d
        out_v[pl.ds(g * L, L)] = acc
        return carry

    lax.fori_loop(0, BPW // L, group, 0)

    pltpu.sync_copy(out_v, out_hbm.at[pl.ds(base, BPW)])


@jax.jit
def kernel(user, item, user_table, item_table):
    mesh = plsc.VectorSubcoreMesh(core_axis_name="c", subcore_axis_name="s",
                                  num_cores=NC, num_subcores=NS)
    f = pl.kernel(
        _mf_body,
        out_type=jax.ShapeDtypeStruct((B,), jnp.float32),
        mesh=mesh,
        compiler_params=pltpu.CompilerParams(needs_layout_passes=False,
                                             use_tc_tiling_on_sc=False),
        scratch_types=[
            pltpu.VMEM((CH, CHB), jnp.int32),
            pltpu.VMEM((CH, CHB), jnp.int32),
            pltpu.VMEM((BPW, D), jnp.float32),
            pltpu.VMEM((BPW, D), jnp.float32),
            pltpu.VMEM((BPW,), jnp.float32),
            pltpu.SemaphoreType.DMA,
        ],
    )
    return f(user.astype(jnp.int32), item.astype(jnp.int32),
             user_table, item_table)


# speedup vs baseline: 1.0065x; 1.0065x over previous
import jax
import jax.numpy as jnp
from jax import lax
from jax.experimental import pallas as pl
from jax.experimental.pallas import tpu as pltpu
from jax.experimental.pallas import tpu_sc as plsc

B = 16384
D = 32
PACK = 128 // D
NT = 1000000 // PACK
NC = 2
NS = 16
L = 16
NW = NC * NS
BPW = B // NW
CHB = 128
CH = BPW // CHB
G = CHB // L


def _mf_body(user_hbm, item_hbm, ut_hbm, it_hbm, out_hbm,
             raw_u, raw_i, hi_u, hi_i, lo_u, lo_i,
             buf_u, buf_i, out_v, sem0, sem1):
    wid = lax.axis_index("s") * NC + lax.axis_index("c")
    base = wid * BPW

    for c in range(CH):
        pltpu.sync_copy(user_hbm.at[pl.ds(base + c * CHB, CHB)], raw_u.at[c])
        pltpu.sync_copy(item_hbm.at[pl.ds(base + c * CHB, CHB)], raw_i.at[c])

    for c in range(CH):
        for t in range(G):
            s = pl.ds(t * L, L)
            u = raw_u[c, s]
            hi_u[c, s] = lax.shift_right_logical(u, 2)
            lo_u[pl.ds(c * CHB + t * L, L)] = lax.shift_left(u & 3, 5)
            v = raw_i[c, s]
            hi_i[c, s] = lax.shift_right_logical(v, 2)
            lo_i[pl.ds(c * CHB + t * L, L)] = lax.shift_left(v & 3, 5)

    sems = (sem0, sem1)

    def fire(c):
        b = c & 1
        pltpu.async_copy(ut_hbm.at[hi_u.at[c]], buf_u.at[b], sems[b])
        pltpu.async_copy(it_hbm.at[hi_i.at[c]], buf_i.at[b], sems[b])

    def drain(c):
        b = c & 1
        pltpu.make_async_copy(ut_hbm.at[hi_u.at[c]], buf_u.at[b], sems[b]).wait()
        pltpu.make_async_copy(it_hbm.at[hi_i.at[c]], buf_i.at[b], sems[b]).wait()

    lane = lax.iota(jnp.int32, L)
    fire(0)
    for c in range(CH):
        if c + 1 < CH:
            fire(c + 1)
        drain(c)
        b = c & 1
        bu = buf_u.at[b]
        bi = buf_i.at[b]

        def group(g, carry):
            rows = g * L + lane
            lu = lo_u[pl.ds(c * CHB + g * L, L)]
            li = lo_i[pl.ds(c * CHB + g * L, L)]
            dcol = lane
            acc = jnp.zeros((L,), jnp.float32)
            for _ in range(D):
                a = plsc.load_gather(bu, [rows, lu + dcol])
                v = plsc.load_gather(bi, [rows, li + dcol])
                acc = acc + a * v
                dcol = (dcol + 1) & (D - 1)
            out_v[pl.ds(c * CHB + g * L, L)] = acc
            return carry

        lax.fori_loop(0, G, group, 0)

    pltpu.sync_copy(out_v, out_hbm.at[pl.ds(base, BPW)])


@jax.jit
def kernel(user, item, user_table, item_table):
    mesh = plsc.VectorSubcoreMesh(core_axis_name="c", subcore_axis_name="s",
                                  num_cores=NC, num_subcores=NS)
    f = pl.kernel(
        _mf_body,
        out_type=jax.ShapeDtypeStruct((B,), jnp.float32),
        mesh=mesh,
        compiler_params=pltpu.CompilerParams(needs_layout_passes=False,
                                             use_tc_tiling_on_sc=True),
        scratch_types=[
            pltpu.VMEM((CH, CHB), jnp.int32),
            pltpu.VMEM((CH, CHB), jnp.int32),
            pltpu.VMEM((CH, CHB), jnp.int32),
            pltpu.VMEM((CH, CHB), jnp.int32),
            pltpu.VMEM((BPW,), jnp.int32),
            pltpu.VMEM((BPW,), jnp.int32),
            pltpu.VMEM((2, CHB, 128), jnp.float32),
            pltpu.VMEM((2, CHB, 128), jnp.float32),
            pltpu.VMEM((BPW,), jnp.float32),
            pltpu.SemaphoreType.DMA,
            pltpu.SemaphoreType.DMA,
        ],
    )
    return f(user.astype(jnp.int32), item.astype(jnp.int32),
             user_table.reshape(NT, 128), item_table.reshape(NT, 128))
